# baseline (speedup 1.0000x reference)
import functools

import jax
import jax.numpy as jnp
from jax import lax
from jax.experimental import pallas as pl
from jax.experimental.pallas import tpu as pltpu
from jax.experimental.pallas import tpu_sc as plsc

D = 64
L = 16
NC, NS = 2, 16
NW = NC * NS
UNROLL = 8


@functools.lru_cache(maxsize=None)
def _make_gather(N, T, V):
    n_per_w = N // NW
    assert N % NW == 0 and n_per_w >= 4
    mesh = plsc.VectorSubcoreMesh(core_axis_name="c", subcore_axis_name="s")

    @functools.partial(
        pl.kernel,
        mesh=mesh,
        out_type=jax.ShapeDtypeStruct((N, T, D), jnp.float32),
        scratch_types=[
            pltpu.VMEM((V * D,), jnp.float32),
            pltpu.VMEM((2, T + L), jnp.int32),
            pltpu.VMEM((2, T, D), jnp.float32),
            pltpu.SemaphoreType.DMA,
            pltpu.SemaphoreType.DMA,
            pltpu.SemaphoreType.DMA,
            pltpu.SemaphoreType.DMA,
            pltpu.SemaphoreType.DMA,
        ],
        compiler_params=pltpu.CompilerParams(use_tc_tiling_on_sc=False),
    )
    def gather(table_hbm, pos_hbm, out_hbm, table_v, idx_v, rows_v,
               tsem, o0, o1, i0, i1):
        osem = (o0, o1)
        isem = (i0, i1)
        wid = lax.axis_index("s") * NC + lax.axis_index("c")
        base = wid * n_per_w

        def idx_load(c, s):
            pltpu.async_copy(
                pos_hbm.at[base + c], idx_v.at[s, pl.ds(0, T)], isem[s]
            )

        def idx_wait(s):
            pltpu.make_async_copy(
                pos_hbm.at[0], idx_v.at[s, pl.ds(0, T)], isem[s]
            ).wait()

        def fire_out(c, s):
            pltpu.async_copy(rows_v.at[s], out_hbm.at[base + c], osem[s])

        def wait_out(s):
            pltpu.make_async_copy(rows_v.at[s], out_hbm.at[0], osem[s]).wait()

        def compute(s):
            def row_group(g, carry):
                r0 = g * 8
                bases = idx_v[s, pl.ds(r0, L)] * D
                for k in range(8):
                    tb = bases[k]
                    p = r0 + k
                    for q in range(0, D, L):
                        rows_v[s, p, pl.ds(q, L)] = table_v[pl.ds(tb + q, L)]
                return carry

            lax.fori_loop(0, T // 8, row_group, 0)

        pltpu.async_copy(table_hbm, table_v, tsem)
        pltpu.make_async_copy(table_hbm, table_v, tsem).wait()

        idx_load(0, 0)
        idx_load(1, 1)
        for c in (0, 1):
            s = c
            idx_wait(s)
            compute(s)
            fire_out(c, s)
            idx_load(c + 2, s)

        def body(i, carry):
            for s in (0, 1):
                c = 2 * i + s
                idx_wait(s)
                wait_out(s)
                compute(s)
                fire_out(c, s)
                idx_load(c + 2, s)
            return carry

        lax.fori_loop(1, n_per_w // 2 - 1, body, 0)

        for c in (n_per_w - 2, n_per_w - 1):
            s = c % 2
            idx_wait(s)
            wait_out(s)
            compute(s)
            fire_out(c, s)
        wait_out(0)
        wait_out(1)

    return gather


def kernel(positions, table):
    N, T = positions.shape
    pos = positions.astype(jnp.int32)
    return _make_gather(N, T, table.shape[0])(table.reshape(-1), pos)


# speedup vs baseline: 1.0245x; 1.0245x over previous
import functools

import jax
import jax.numpy as jnp
from jax import lax
from jax.experimental import pallas as pl
from jax.experimental.pallas import tpu as pltpu
from jax.experimental.pallas import tpu_sc as plsc

D = 64
L = 16
NC, NS = 2, 16
NW = NC * NS
CHUNK = 512
OROWS = CHUNK * D // 128


@functools.lru_cache(maxsize=None)
def _make_gather(B, V):
    b_per_w = B // NW
    nchunks = b_per_w // CHUNK
    assert b_per_w % CHUNK == 0 and nchunks % 2 == 0 and nchunks >= 6
    mesh = plsc.VectorSubcoreMesh(core_axis_name="c", subcore_axis_name="s")

    @functools.partial(
        pl.kernel,
        mesh=mesh,
        out_type=jax.ShapeDtypeStruct((B * D // 128, 128), jnp.float32),
        scratch_types=[
            pltpu.VMEM((V * D,), jnp.float32),
            pltpu.VMEM((CHUNK + L,), jnp.int32),
            pltpu.VMEM((CHUNK + L,), jnp.int32),
            pltpu.VMEM((OROWS, 128), jnp.float32),
            pltpu.VMEM((OROWS, 128), jnp.float32),
            pltpu.SemaphoreType.DMA,
            pltpu.SemaphoreType.DMA,
            pltpu.SemaphoreType.DMA,
            pltpu.SemaphoreType.DMA,
            pltpu.SemaphoreType.DMA,
        ],
        compiler_params=pltpu.CompilerParams(use_tc_tiling_on_sc=True),
    )
    def gather(table_hbm, pos_hbm, out_hbm, table_v, ix0, ix1, rw0, rw1,
               tsem, o0, o1, i0, i1):
        idx_v = (ix0, ix1)
        rows_v = (rw0, rw1)
        osem = (o0, o1)
        isem = (i0, i1)
        wid = lax.axis_index("s") * NC + lax.axis_index("c")
        base = wid * b_per_w
        obase = base * D // 128

        def idx_load(c, s):
            pltpu.async_copy(
                pos_hbm.at[pl.ds(pl.multiple_of(base + c * CHUNK, CHUNK), CHUNK)],
                idx_v[s].at[pl.ds(0, CHUNK)],
                isem[s],
            )

        def idx_wait(s):
            pltpu.make_async_copy(
                pos_hbm.at[pl.ds(0, CHUNK)],
                idx_v[s].at[pl.ds(0, CHUNK)],
                isem[s],
            ).wait()

        def fire_out(c, s):
            pltpu.async_copy(
                rows_v[s],
                out_hbm.at[pl.ds(pl.multiple_of(obase + c * OROWS, OROWS), OROWS)],
                osem[s],
            )

        def wait_out(s):
            pltpu.make_async_copy(
                rows_v[s], out_hbm.at[pl.ds(0, OROWS)], osem[s]
            ).wait()

        def compute(s):
            def row_group(g, carry):
                p0 = pl.multiple_of(g * L, L)
                bases = idx_v[s][pl.ds(p0, L)] * D
                blk = rows_v[s].at[pl.ds(pl.multiple_of(g * 8, 8), 8)]
                for k in range(L):
                    tb = pl.multiple_of(bases[k], D)
                    half = (k % 2) * D
                    for q in range(0, D, L):
                        blk[k // 2, pl.ds(half + q, L)] = (
                            table_v[pl.ds(tb + q, L)]
                        )
                return carry

            lax.fori_loop(0, CHUNK // L, row_group, 0)

        pltpu.async_copy(table_hbm, table_v, tsem)
        pltpu.make_async_copy(table_hbm, table_v, tsem).wait()

        idx_load(0, 0)
        idx_load(1, 1)
        for c in (0, 1):
            s = c
            idx_wait(s)
            compute(s)
            fire_out(c, s)
            idx_load(c + 2, s)

        def body(i, carry):
            for s in (0, 1):
                c = 2 * i + s
                idx_wait(s)
                wait_out(s)
                compute(s)
                fire_out(c, s)
                idx_load(c + 2, s)
            return carry

        lax.fori_loop(1, nchunks // 2 - 1, body, 0)

        for c in (nchunks - 2, nchunks - 1):
            s = c % 2
            idx_wait(s)
            wait_out(s)
            compute(s)
            fire_out(c, s)
        wait_out(0)
        wait_out(1)

    return gather


def kernel(positions, table):
    N, T = positions.shape
    pos_flat = positions.reshape(-1).astype(jnp.int32)
    B = pos_flat.shape[0]
    out = _make_gather(B, table.shape[0])(table.reshape(-1), pos_flat)
    return out.reshape(N, T, D)


# speedup vs baseline: 1.3343x; 1.3024x over previous
import functools

import jax
import jax.numpy as jnp
from jax import lax
from jax.experimental import pallas as pl
from jax.experimental.pallas import tpu as pltpu
from jax.experimental.pallas import tpu_sc as plsc

D = 64
L = 16
NC, NS = 2, 16
NW = NC * NS


@functools.lru_cache(maxsize=None)
def _make_gather(N, T, V):
    n_per_w = N // NW
    assert N % NW == 0 and n_per_w % 2 == 0 and n_per_w >= 6
    ngroups, tail = divmod(T, L)
    assert tail % 8 == 0
    mesh = plsc.VectorSubcoreMesh(core_axis_name="c", subcore_axis_name="s")

    @functools.partial(
        pl.kernel,
        mesh=mesh,
        out_type=jax.ShapeDtypeStruct((N, T, D), jnp.float32),
        scratch_types=[
            pltpu.VMEM((V * D,), jnp.float32),
            pltpu.VMEM((T + L,), jnp.int32),
            pltpu.VMEM((T + L,), jnp.int32),
            pltpu.VMEM((T, D), jnp.float32),
            pltpu.VMEM((T, D), jnp.float32),
            pltpu.SemaphoreType.DMA,
            pltpu.SemaphoreType.DMA,
            pltpu.SemaphoreType.DMA,
            pltpu.SemaphoreType.DMA,
            pltpu.SemaphoreType.DMA,
        ],
        compiler_params=pltpu.CompilerParams(use_tc_tiling_on_sc=True),
    )
    def gather(table_hbm, pos_hbm, out_hbm, table_v, ix0, ix1, rw0, rw1,
               tsem, o0, o1, i0, i1):
        idx_v = (ix0, ix1)
        rows_v = (rw0, rw1)
        osem = (o0, o1)
        isem = (i0, i1)
        wid = lax.axis_index("s") * NC + lax.axis_index("c")
        base = wid * n_per_w

        def idx_load(c, s):
            pltpu.async_copy(
                pos_hbm.at[pl.ds(pl.multiple_of((base + c) * T, T), T)],
                idx_v[s].at[pl.ds(0, T)],
                isem[s],
            )

        def idx_wait(s):
            pltpu.make_async_copy(
                pos_hbm.at[pl.ds(0, T)], idx_v[s].at[pl.ds(0, T)], isem[s]
            ).wait()

        def fire_out(c, s):
            pltpu.async_copy(rows_v[s], out_hbm.at[base + c], osem[s])

        def wait_out(s):
            pltpu.make_async_copy(rows_v[s], out_hbm.at[0], osem[s]).wait()

        def compute(s):
            def do_rows(p0, nrows):
                bases = idx_v[s][pl.ds(p0, L)] * D
                blk = rows_v[s].at[pl.ds(p0, nrows)]
                for k in range(nrows):
                    tb = pl.multiple_of(bases[k], D)
                    for q in range(0, D, L):
                        blk[k, pl.ds(q, L)] = table_v[pl.ds(tb + q, L)]

            def group(g, carry):
                do_rows(pl.multiple_of(g * L, L), L)
                return carry

            lax.fori_loop(0, ngroups, group, 0)
            if tail:
                do_rows(ngroups * L, tail)

        pltpu.async_copy(table_hbm, table_v, tsem)
        pltpu.make_async_copy(table_hbm, table_v, tsem).wait()

        idx_load(0, 0)
        idx_load(1, 1)
        for c in (0, 1):
            s = c
            idx_wait(s)
            compute(s)
            fire_out(c, s)
            idx_load(c + 2, s)

        def body(i, carry):
            for s in (0, 1):
                c = 2 * i + s
                idx_wait(s)
                wait_out(s)
                compute(s)
                fire_out(c, s)
                idx_load(c + 2, s)
            return carry

        lax.fori_loop(1, n_per_w // 2 - 1, body, 0)

        for c in (n_per_w - 2, n_per_w - 1):
            s = c % 2
            idx_wait(s)
            wait_out(s)
            compute(s)
            fire_out(c, s)
        wait_out(0)
        wait_out(1)

    return gather


def kernel(positions, table):
    N, T = positions.shape
    pos_flat = positions.reshape(-1).astype(jnp.int32)
    return _make_gather(N, T, table.shape[0])(table.reshape(-1), pos_flat)
